# baseline (speedup 1.0000x reference)
import functools
import math

import jax
import jax.numpy as jnp
from jax.experimental import pallas as pl
from jax.experimental.pallas import tpu as pltpu


COMMIT_W = 0.25
NBINS = 2048
BIG = 3.0e38


def _fwd_block(x_ref, we0, be0, we1, be1, we2, be2,
               wd0, bd0, wd1, bd1, wd2, bd2, cb_ref,
               stats_ref, keys_ref, acc_ref, macc_ref,
               cbh_ref, cbm_ref, cbl_ref,
               *, n_levels, n_codes):
    pid = pl.program_id(0)
    x = x_ref[...]
    bb = x.shape[0]

    @pl.when(pid == 0)
    def _():
        cbf = cb_ref[...]
        u = jax.lax.bitcast_convert_type(cbf, jnp.uint32)
        hi = jax.lax.bitcast_convert_type(u & jnp.uint32(0xFFFF0000),
                                          jnp.float32)
        r1 = cbf - hi
        u1 = jax.lax.bitcast_convert_type(r1, jnp.uint32)
        mid = jax.lax.bitcast_convert_type(u1 & jnp.uint32(0xFFFF0000),
                                           jnp.float32)
        cbh_ref[...] = hi.astype(jnp.bfloat16)
        cbm_ref[...] = mid.astype(jnp.bfloat16)
        cbl_ref[...] = (r1 - mid).astype(jnp.bfloat16)

    h = jnp.maximum(jnp.dot(x, we0[...], preferred_element_type=jnp.float32)
                    + be0[...], 0.0)
    h = jnp.maximum(jnp.dot(h, we1[...], preferred_element_type=jnp.float32)
                    + be1[...], 0.0)
    res = jnp.dot(h, we2[...], preferred_element_type=jnp.float32) + be2[...]
    z_enc = res

    def vq_chain(res_h, row0):
        hb = res_h.shape[0]
        iota = jax.lax.broadcasted_iota(jnp.int32, (hb, n_codes), 1)
        key = jnp.zeros((hb, 1), jnp.int32)
        ql = jnp.float32(0.0)
        zq = jnp.zeros_like(res_h)
        for l in range(n_levels):
            cb = cb_ref[l]
            rn = jnp.sum(res_h * res_h, axis=-1, keepdims=True)
            scores = jax.lax.dot_general(res_h, cb, (((1,), (1,)), ((), ())),
                                         preferred_element_type=jnp.float32)
            cn = jnp.sum(cb * cb, axis=-1)[None, :]
            d = rn - 2.0 * scores + cn
            ids = jnp.argmin(d, axis=-1, keepdims=True).astype(jnp.int32)
            oh = (iota == ids).astype(jnp.bfloat16)
            parts = []
            for part_ref in (cbh_ref, cbm_ref, cbl_ref):
                parts.append(jax.lax.dot_general(
                    oh, part_ref[l], (((1,), (0,)), ((), ())),
                    preferred_element_type=jnp.float32))
            emb_q = (parts[0] + parts[1]) + parts[2]
            t = emb_q - res_h
            emb = res_h + t
            stats_ref[row0:row0 + hb, l:l + 1] = jnp.sqrt(
                jnp.sum(emb * emb, axis=-1, keepdims=True))
            ql = ql + jnp.sum(t * t)
            res_h = res_h - emb
            zq = zq + emb
            key = key * n_codes + ids
        keys_ref[row0:row0 + hb, 0:1] = key
        return ql, zq, res_h

    nchain = 8
    hb = bb // nchain
    ql_sum = jnp.float32(0.0)
    zqs, rfs = [], []
    for ci in range(nchain):
        qlc, zqc, rfc = vq_chain(res[ci * hb:(ci + 1) * hb], ci * hb)
        ql_sum = ql_sum + qlc
        zqs.append(zqc)
        rfs.append(rfc)
    z_q = jnp.concatenate(zqs, axis=0)
    res = jnp.concatenate(rfs, axis=0)

    g = jnp.maximum(jnp.dot(z_q, wd0[...], preferred_element_type=jnp.float32)
                    + bd0[...], 0.0)
    g = jnp.maximum(jnp.dot(g, wd1[...], preferred_element_type=jnp.float32)
                    + bd1[...], 0.0)
    g = jnp.dot(g, wd2[...], preferred_element_type=jnp.float32) + bd2[...]
    for _ in range(2):
        nrm = jnp.sqrt(jnp.sum(g * g, axis=-1, keepdims=True))
        g = g / jnp.maximum(nrm, 1e-12)

    dx = g - x
    recon = jnp.sum(dx * dx, axis=-1, keepdims=True)
    rerr = jnp.sqrt(recon)
    stats_ref[:, 3:4] = rerr
    nx = jnp.sqrt(jnp.sum(x * x, axis=-1, keepdims=True))
    ng = jnp.sqrt(jnp.sum(g * g, axis=-1, keepdims=True))
    cos = jnp.sum(x * g, axis=-1, keepdims=True) / (nx * ng + 1e-8)
    dq = z_enc - z_q
    resn = jnp.sqrt(jnp.sum(dq * dq, axis=-1, keepdims=True))

    li = jax.lax.broadcasted_iota(jnp.int32, (1, 8), 1)
    vals = (jnp.where(li == 0, jnp.sum(recon), 0.0)
            + jnp.where(li == 1, ql_sum, 0.0)
            + jnp.where(li == 2, jnp.sum(cos), 0.0)
            + jnp.where(li == 3, jnp.sum(resn), 0.0))
    mvals = jnp.where(li == 0, jnp.min(rerr),
                      jnp.where(li == 1, -jnp.max(rerr), BIG))

    @pl.when(pid == 0)
    def _():
        acc_ref[...] = jnp.zeros_like(acc_ref)
        macc_ref[...] = jnp.full_like(macc_ref, BIG)

    acc_ref[...] += vals
    macc_ref[...] = jnp.minimum(macc_ref[...], mvals)


def _stats_block(k_row_ref, stats_ref, keys_ref, macc_ref,
                 out_ref, hacc_ref, *, rb, n, nb, targets):
    pid = pl.program_id(0)
    nblk = n // rb
    base = pid * rb
    k_col = keys_ref[pl.ds(base, rb), 0:1]
    irow = base + jax.lax.broadcasted_iota(jnp.int32, (rb, 1), 0)

    k_row = k_row_ref[...]
    jiota = jax.lax.broadcasted_iota(jnp.int32, (rb, n), 1)
    dup_after = jnp.logical_and(k_row == k_col, jiota > irow)
    has_dup = jnp.max(dup_after.astype(jnp.float32), axis=-1, keepdims=True)
    distinct_part = jnp.sum(1.0 - has_dup)

    e_col = stats_ref[pl.ds(base, rb), 3:4]
    mn = macc_ref[0, 0]
    mx = -macc_ref[0, 1]
    w = jnp.maximum((mx - mn) * (1.0 / nb), 1e-30)
    binid = jnp.clip(jnp.floor((e_col - mn) / w).astype(jnp.int32), 0, nb - 1)
    rid = binid // 128
    lid = binid - rid * 128
    arow = (jax.lax.broadcasted_iota(jnp.int32, (rb, nb // 128), 1)
            == rid).astype(jnp.bfloat16)
    alane = (jax.lax.broadcasted_iota(jnp.int32, (rb, 128), 1)
             == lid).astype(jnp.bfloat16)
    hist = jax.lax.dot_general(arow, alane, (((0,), (0,)), ((), ())),
                               preferred_element_type=jnp.float32)

    li = jax.lax.broadcasted_iota(jnp.int32, (1, 8), 1)

    @pl.when(pid == 0)
    def _():
        out_ref[...] = jnp.zeros_like(out_ref)
        hacc_ref[...] = jnp.zeros_like(hacc_ref)

    out_ref[...] += jnp.where(li == 6, distinct_part, 0.0)
    hacc_ref[...] += hist

    @pl.when(pid == nblk - 1)
    def _():
        h2 = hacc_ref[...]
        ut = (jax.lax.broadcasted_iota(jnp.int32, (128, 128), 0)
              <= jax.lax.broadcasted_iota(jnp.int32, (128, 128), 1)
              ).astype(jnp.float32)
        cum_row = jax.lax.dot_general(h2, ut, (((1,), (0,)), ((), ())),
                                      preferred_element_type=jnp.float32,
                                      precision=jax.lax.Precision.HIGHEST)
        tot = jnp.sum(h2, axis=1, keepdims=True)
        m = nb // 128
        st = (jax.lax.broadcasted_iota(jnp.int32, (m, m), 0)
              < jax.lax.broadcasted_iota(jnp.int32, (m, m), 1)
              ).astype(jnp.float32)
        prefix = jax.lax.dot_general(st, tot, (((0,), (0,)), ((), ())),
                                     preferred_element_type=jnp.float32,
                                     precision=jax.lax.Precision.HIGHEST)
        cum = cum_row + prefix
        qv = jnp.zeros((1, 8), jnp.float32)
        for slot, tgt in enumerate(targets):
            bidx = jnp.sum((cum <= float(tgt)).astype(jnp.float32))
            qv = qv + jnp.where(li == slot, bidx, 0.0)
        out_ref[...] += qv


def kernel(x, We0, be0, We1, be1, We2, be2, Wd0, bd0, Wd1, bd1, Wd2, bd2,
           codebooks, gumbel_t):
    B, D = x.shape
    L, K, E = codebooks.shape
    BB = 1024
    grid_a = B // BB

    be0r, be1r, be2r = be0[None, :], be1[None, :], be2[None, :]
    bd0r, bd1r, bd2r = bd0[None, :], bd1[None, :], bd2[None, :]

    full = lambda arr: pl.BlockSpec(arr.shape, lambda i: (0,) * arr.ndim)
    fwd = functools.partial(_fwd_block, n_levels=L, n_codes=K)
    stats, keys, acc, macc = pl.pallas_call(
        fwd,
        grid=(grid_a,),
        in_specs=[
            pl.BlockSpec((BB, D), lambda i: (i, 0)),
            full(We0), full(be0r), full(We1), full(be1r),
            full(We2), full(be2r),
            full(Wd0), full(bd0r), full(Wd1), full(bd1r),
            full(Wd2), full(bd2r),
            full(codebooks),
        ],
        out_specs=[
            pl.BlockSpec((BB, 8), lambda i: (i, 0)),
            pl.BlockSpec((BB, 8), lambda i: (i, 0)),
            pl.BlockSpec((1, 8), lambda i: (0, 0)),
            pl.BlockSpec((1, 8), lambda i: (0, 0)),
        ],
        out_shape=[
            jax.ShapeDtypeStruct((B, 8), jnp.float32),
            jax.ShapeDtypeStruct((B, 8), jnp.int32),
            jax.ShapeDtypeStruct((1, 8), jnp.float32),
            jax.ShapeDtypeStruct((1, 8), jnp.float32),
        ],
        scratch_shapes=[
            pltpu.VMEM((L, K, E), jnp.bfloat16),
            pltpu.VMEM((L, K, E), jnp.bfloat16),
            pltpu.VMEM((L, K, E), jnp.bfloat16),
        ],
    )(x, We0, be0r, We1, be1r, We2, be2r,
      Wd0, bd0r, Wd1, bd1r, Wd2, bd2r, codebooks)

    qspec = []
    ranks = []
    for q in (0.5, 0.9, 0.99):
        pos = q * (B - 1)
        lo_r = int(math.floor(pos))
        frac = pos - lo_r
        qspec.append(frac)
        ranks.extend([lo_r, lo_r + 1])

    k_row = keys[:, 0].reshape(1, B)
    RB = 256
    sb = functools.partial(_stats_block, rb=RB, n=B, nb=NBINS,
                           targets=tuple(ranks))
    qacc, _hist = pl.pallas_call(
        sb,
        grid=(B // RB,),
        in_specs=[full(k_row), full(stats), full(keys), full(macc)],
        out_specs=[pl.BlockSpec((1, 8), lambda i: (0, 0)),
                   pl.BlockSpec((NBINS // 128, 128), lambda i: (0, 0))],
        out_shape=[jax.ShapeDtypeStruct((1, 8), jnp.float32),
                   jax.ShapeDtypeStruct((NBINS // 128, 128), jnp.float32)],
    )(k_row, stats, keys, macc)

    s_recon, s_ql, s_cos, s_resn = acc[0, 0], acc[0, 1], acc[0, 2], acc[0, 3]
    mean_recon = s_recon / B
    rq_l = (1.0 + COMMIT_W) * s_ql / B
    loss = mean_recon + rq_l
    cosine_sim = s_cos / B
    rmse = jnp.sqrt(s_recon / (B * D))
    quantization_error = s_resn / B
    embs_norm = stats[:, :L].T
    p_unique_ids = qacc[0, 6] / B

    mn = macc[0, 0]
    mx = -macc[0, 1]
    w = jnp.maximum((mx - mn) * (1.0 / NBINS), 1e-30)
    qs = []
    for i, frac in enumerate(qspec):
        v_lo = mn + w * (qacc[0, 2 * i] + 0.5)
        v_hi = mn + w * (qacc[0, 2 * i + 1] + 0.5)
        qs.append(v_lo * (1.0 - jnp.float32(frac)) + v_hi * jnp.float32(frac))
    p50, p90, p99 = qs

    return (loss, mean_recon, rq_l, embs_norm, p_unique_ids, cosine_sim,
            rmse, quantization_error, p50, p90, p99)


# speedup vs baseline: 1.2315x; 1.2315x over previous
import functools
import math

import jax
import jax.numpy as jnp
from jax.experimental import pallas as pl
from jax.experimental.pallas import tpu as pltpu


COMMIT_W = 0.25
NBINS = 2048
BIG = 3.0e38


def _fwd_block(x_ref, we0, be0, we1, be1, we2, be2,
               wd0, bd0, wd1, bd1, wd2, bd2, cb_ref,
               stats_ref, keys_ref, acc_ref, macc_ref,
               cbh_ref, cbm_ref, cbl_ref,
               *, n_levels, n_codes):
    pid = pl.program_id(0)
    x = x_ref[...]
    bb = x.shape[0]

    @pl.when(pid == 0)
    def _():
        cbf = cb_ref[...]
        u = jax.lax.bitcast_convert_type(cbf, jnp.uint32)
        hi = jax.lax.bitcast_convert_type(u & jnp.uint32(0xFFFF0000),
                                          jnp.float32)
        r1 = cbf - hi
        u1 = jax.lax.bitcast_convert_type(r1, jnp.uint32)
        mid = jax.lax.bitcast_convert_type(u1 & jnp.uint32(0xFFFF0000),
                                           jnp.float32)
        cbh_ref[...] = hi.astype(jnp.bfloat16)
        cbm_ref[...] = mid.astype(jnp.bfloat16)
        cbl_ref[...] = (r1 - mid).astype(jnp.bfloat16)

    h = jnp.maximum(jnp.dot(x, we0[...], preferred_element_type=jnp.float32)
                    + be0[...], 0.0)
    h = jnp.maximum(jnp.dot(h, we1[...], preferred_element_type=jnp.float32)
                    + be1[...], 0.0)
    res = jnp.dot(h, we2[...], preferred_element_type=jnp.float32) + be2[...]
    z_enc = res

    def vq_chain(res_h, row0):
        hb = res_h.shape[0]
        iota = jax.lax.broadcasted_iota(jnp.int32, (hb, n_codes), 1)
        key = jnp.zeros((hb, 1), jnp.int32)
        ql = jnp.float32(0.0)
        zq = jnp.zeros_like(res_h)
        for l in range(n_levels):
            cb = cb_ref[l]
            rn = jnp.sum(res_h * res_h, axis=-1, keepdims=True)
            scores = jax.lax.dot_general(res_h, cb, (((1,), (1,)), ((), ())),
                                         preferred_element_type=jnp.float32)
            cn = jnp.sum(cb * cb, axis=-1)[None, :]
            d = rn - 2.0 * scores + cn
            ids = jnp.argmin(d, axis=-1, keepdims=True).astype(jnp.int32)
            oh = (iota == ids).astype(jnp.bfloat16)
            parts = []
            for part_ref in (cbh_ref, cbm_ref, cbl_ref):
                parts.append(jax.lax.dot_general(
                    oh, part_ref[l], (((1,), (0,)), ((), ())),
                    preferred_element_type=jnp.float32))
            emb_q = (parts[0] + parts[1]) + parts[2]
            t = emb_q - res_h
            emb = res_h + t
            stats_ref[row0:row0 + hb, l:l + 1] = jnp.sqrt(
                jnp.sum(emb * emb, axis=-1, keepdims=True))
            ql = ql + jnp.sum(t * t)
            res_h = res_h - emb
            zq = zq + emb
            key = key * n_codes + ids
        keys_ref[row0:row0 + hb, 0:1] = key
        return ql, zq, res_h

    nchain = 4
    hb = bb // nchain
    ql_sum = jnp.float32(0.0)
    zqs, rfs = [], []
    for ci in range(nchain):
        qlc, zqc, rfc = vq_chain(res[ci * hb:(ci + 1) * hb], ci * hb)
        ql_sum = ql_sum + qlc
        zqs.append(zqc)
        rfs.append(rfc)
    z_q = jnp.concatenate(zqs, axis=0)
    res = jnp.concatenate(rfs, axis=0)

    g = jnp.maximum(jnp.dot(z_q, wd0[...], preferred_element_type=jnp.float32)
                    + bd0[...], 0.0)
    g = jnp.maximum(jnp.dot(g, wd1[...], preferred_element_type=jnp.float32)
                    + bd1[...], 0.0)
    g = jnp.dot(g, wd2[...], preferred_element_type=jnp.float32) + bd2[...]
    for _ in range(2):
        nrm = jnp.sqrt(jnp.sum(g * g, axis=-1, keepdims=True))
        g = g / jnp.maximum(nrm, 1e-12)

    dx = g - x
    recon = jnp.sum(dx * dx, axis=-1, keepdims=True)
    rerr = jnp.sqrt(recon)
    stats_ref[:, 3:4] = rerr
    nx = jnp.sqrt(jnp.sum(x * x, axis=-1, keepdims=True))
    ng = jnp.sqrt(jnp.sum(g * g, axis=-1, keepdims=True))
    cos = jnp.sum(x * g, axis=-1, keepdims=True) / (nx * ng + 1e-8)
    dq = z_enc - z_q
    resn = jnp.sqrt(jnp.sum(dq * dq, axis=-1, keepdims=True))

    li = jax.lax.broadcasted_iota(jnp.int32, (1, 8), 1)
    vals = (jnp.where(li == 0, jnp.sum(recon), 0.0)
            + jnp.where(li == 1, ql_sum, 0.0)
            + jnp.where(li == 2, jnp.sum(cos), 0.0)
            + jnp.where(li == 3, jnp.sum(resn), 0.0))
    mvals = jnp.where(li == 0, jnp.min(rerr),
                      jnp.where(li == 1, -jnp.max(rerr), BIG))

    @pl.when(pid == 0)
    def _():
        acc_ref[...] = jnp.zeros_like(acc_ref)
        macc_ref[...] = jnp.full_like(macc_ref, BIG)

    acc_ref[...] += vals
    macc_ref[...] = jnp.minimum(macc_ref[...], mvals)


def _stats_block(k_row_ref, stats_ref, keys_ref, macc_ref,
                 out_ref, hacc_ref, *, rb, n, nb, targets):
    pid = pl.program_id(0)
    nblk = n // rb
    base = pid * rb
    k_col = keys_ref[pl.ds(base, rb), 0:1]
    irow = base + jax.lax.broadcasted_iota(jnp.int32, (rb, 1), 0)

    k_row = k_row_ref[...]
    jiota = jax.lax.broadcasted_iota(jnp.int32, (rb, n), 1)
    dup_after = jnp.logical_and(k_row == k_col, jiota > irow)
    has_dup = jnp.max(dup_after.astype(jnp.float32), axis=-1, keepdims=True)
    distinct_part = jnp.sum(1.0 - has_dup)

    e_col = stats_ref[pl.ds(base, rb), 3:4]
    mn = macc_ref[0, 0]
    mx = -macc_ref[0, 1]
    w = jnp.maximum((mx - mn) * (1.0 / nb), 1e-30)
    binid = jnp.clip(jnp.floor((e_col - mn) / w).astype(jnp.int32), 0, nb - 1)
    rid = binid // 128
    lid = binid - rid * 128
    arow = (jax.lax.broadcasted_iota(jnp.int32, (rb, nb // 128), 1)
            == rid).astype(jnp.bfloat16)
    alane = (jax.lax.broadcasted_iota(jnp.int32, (rb, 128), 1)
             == lid).astype(jnp.bfloat16)
    hist = jax.lax.dot_general(arow, alane, (((0,), (0,)), ((), ())),
                               preferred_element_type=jnp.float32)

    li = jax.lax.broadcasted_iota(jnp.int32, (1, 8), 1)

    @pl.when(pid == 0)
    def _():
        out_ref[...] = jnp.zeros_like(out_ref)
        hacc_ref[...] = jnp.zeros_like(hacc_ref)

    out_ref[...] += jnp.where(li == 6, distinct_part, 0.0)
    hacc_ref[...] += hist

    @pl.when(pid == nblk - 1)
    def _():
        h2 = hacc_ref[...]
        ut = (jax.lax.broadcasted_iota(jnp.int32, (128, 128), 0)
              <= jax.lax.broadcasted_iota(jnp.int32, (128, 128), 1)
              ).astype(jnp.float32)
        cum_row = jax.lax.dot_general(h2, ut, (((1,), (0,)), ((), ())),
                                      preferred_element_type=jnp.float32,
                                      precision=jax.lax.Precision.HIGHEST)
        tot = jnp.sum(h2, axis=1, keepdims=True)
        m = nb // 128
        st = (jax.lax.broadcasted_iota(jnp.int32, (m, m), 0)
              < jax.lax.broadcasted_iota(jnp.int32, (m, m), 1)
              ).astype(jnp.float32)
        prefix = jax.lax.dot_general(st, tot, (((0,), (0,)), ((), ())),
                                     preferred_element_type=jnp.float32,
                                     precision=jax.lax.Precision.HIGHEST)
        cum = cum_row + prefix
        qv = jnp.zeros((1, 8), jnp.float32)
        for slot, tgt in enumerate(targets):
            bidx = jnp.sum((cum <= float(tgt)).astype(jnp.float32))
            qv = qv + jnp.where(li == slot, bidx, 0.0)
        out_ref[...] += qv


def kernel(x, We0, be0, We1, be1, We2, be2, Wd0, bd0, Wd1, bd1, Wd2, bd2,
           codebooks, gumbel_t):
    B, D = x.shape
    L, K, E = codebooks.shape
    BB = 1024
    grid_a = B // BB

    be0r, be1r, be2r = be0[None, :], be1[None, :], be2[None, :]
    bd0r, bd1r, bd2r = bd0[None, :], bd1[None, :], bd2[None, :]

    full = lambda arr: pl.BlockSpec(arr.shape, lambda i: (0,) * arr.ndim)
    fwd = functools.partial(_fwd_block, n_levels=L, n_codes=K)
    stats, keys, acc, macc = pl.pallas_call(
        fwd,
        grid=(grid_a,),
        in_specs=[
            pl.BlockSpec((BB, D), lambda i: (i, 0)),
            full(We0), full(be0r), full(We1), full(be1r),
            full(We2), full(be2r),
            full(Wd0), full(bd0r), full(Wd1), full(bd1r),
            full(Wd2), full(bd2r),
            full(codebooks),
        ],
        out_specs=[
            pl.BlockSpec((BB, 8), lambda i: (i, 0)),
            pl.BlockSpec((BB, 8), lambda i: (i, 0)),
            pl.BlockSpec((1, 8), lambda i: (0, 0)),
            pl.BlockSpec((1, 8), lambda i: (0, 0)),
        ],
        out_shape=[
            jax.ShapeDtypeStruct((B, 8), jnp.float32),
            jax.ShapeDtypeStruct((B, 8), jnp.int32),
            jax.ShapeDtypeStruct((1, 8), jnp.float32),
            jax.ShapeDtypeStruct((1, 8), jnp.float32),
        ],
        scratch_shapes=[
            pltpu.VMEM((L, K, E), jnp.bfloat16),
            pltpu.VMEM((L, K, E), jnp.bfloat16),
            pltpu.VMEM((L, K, E), jnp.bfloat16),
        ],
    )(x, We0, be0r, We1, be1r, We2, be2r,
      Wd0, bd0r, Wd1, bd1r, Wd2, bd2r, codebooks)

    qspec = []
    ranks = []
    for q in (0.5, 0.9, 0.99):
        pos = q * (B - 1)
        lo_r = int(math.floor(pos))
        frac = pos - lo_r
        qspec.append(frac)
        ranks.extend([lo_r, lo_r + 1])

    k_row = keys[:, 0].reshape(1, B)
    RB = 512
    sb = functools.partial(_stats_block, rb=RB, n=B, nb=NBINS,
                           targets=tuple(ranks))
    qacc, _hist = pl.pallas_call(
        sb,
        grid=(B // RB,),
        in_specs=[full(k_row), full(stats), full(keys), full(macc)],
        out_specs=[pl.BlockSpec((1, 8), lambda i: (0, 0)),
                   pl.BlockSpec((NBINS // 128, 128), lambda i: (0, 0))],
        out_shape=[jax.ShapeDtypeStruct((1, 8), jnp.float32),
                   jax.ShapeDtypeStruct((NBINS // 128, 128), jnp.float32)],
    )(k_row, stats, keys, macc)

    s_recon, s_ql, s_cos, s_resn = acc[0, 0], acc[0, 1], acc[0, 2], acc[0, 3]
    mean_recon = s_recon / B
    rq_l = (1.0 + COMMIT_W) * s_ql / B
    loss = mean_recon + rq_l
    cosine_sim = s_cos / B
    rmse = jnp.sqrt(s_recon / (B * D))
    quantization_error = s_resn / B
    embs_norm = stats[:, :L].T
    p_unique_ids = qacc[0, 6] / B

    mn = macc[0, 0]
    mx = -macc[0, 1]
    w = jnp.maximum((mx - mn) * (1.0 / NBINS), 1e-30)
    qs = []
    for i, frac in enumerate(qspec):
        v_lo = mn + w * (qacc[0, 2 * i] + 0.5)
        v_hi = mn + w * (qacc[0, 2 * i + 1] + 0.5)
        qs.append(v_lo * (1.0 - jnp.float32(frac)) + v_hi * jnp.float32(frac))
    p50, p90, p99 = qs

    return (loss, mean_recon, rq_l, embs_norm, p_unique_ids, cosine_sim,
            rmse, quantization_error, p50, p90, p99)


# speedup vs baseline: 1.2457x; 1.0116x over previous
import functools
import math

import jax
import jax.numpy as jnp
from jax.experimental import pallas as pl
from jax.experimental.pallas import tpu as pltpu


COMMIT_W = 0.25
NBINS = 2048
BIG = 3.0e38


def _fwd_block(x_ref, we0, be0, we1, be1, we2, be2,
               wd0, bd0, wd1, bd1, wd2, bd2, cb_ref,
               stats_ref, keys_ref, acc_ref, macc_ref,
               cbh_ref, cbm_ref, cbl_ref,
               *, n_levels, n_codes):
    pid = pl.program_id(0)
    x = x_ref[...]
    bb = x.shape[0]

    @pl.when(pid == 0)
    def _():
        cbf = cb_ref[...]
        u = jax.lax.bitcast_convert_type(cbf, jnp.uint32)
        hi = jax.lax.bitcast_convert_type(u & jnp.uint32(0xFFFF0000),
                                          jnp.float32)
        r1 = cbf - hi
        u1 = jax.lax.bitcast_convert_type(r1, jnp.uint32)
        mid = jax.lax.bitcast_convert_type(u1 & jnp.uint32(0xFFFF0000),
                                           jnp.float32)
        cbh_ref[...] = hi.astype(jnp.bfloat16)
        cbm_ref[...] = mid.astype(jnp.bfloat16)
        cbl_ref[...] = (r1 - mid).astype(jnp.bfloat16)

    h = jnp.maximum(jnp.dot(x, we0[...], preferred_element_type=jnp.float32)
                    + be0[...], 0.0)
    h = jnp.maximum(jnp.dot(h, we1[...], preferred_element_type=jnp.float32)
                    + be1[...], 0.0)
    res = jnp.dot(h, we2[...], preferred_element_type=jnp.float32) + be2[...]
    z_enc = res

    def vq_chain(res_h, row0):
        hb = res_h.shape[0]
        iota = jax.lax.broadcasted_iota(jnp.int32, (hb, n_codes), 1)
        key = jnp.zeros((hb, 1), jnp.int32)
        ql = jnp.float32(0.0)
        zq = jnp.zeros_like(res_h)
        for l in range(n_levels):
            cb = cb_ref[l]
            rn = jnp.sum(res_h * res_h, axis=-1, keepdims=True)
            scores = jax.lax.dot_general(res_h, cb, (((1,), (1,)), ((), ())),
                                         preferred_element_type=jnp.float32)
            cn = jnp.sum(cb * cb, axis=-1)[None, :]
            d = rn - 2.0 * scores + cn
            ids = jnp.argmin(d, axis=-1, keepdims=True).astype(jnp.int32)
            oh = (iota == ids).astype(jnp.bfloat16)
            parts = []
            for part_ref in (cbh_ref, cbm_ref, cbl_ref):
                parts.append(jax.lax.dot_general(
                    oh, part_ref[l], (((1,), (0,)), ((), ())),
                    preferred_element_type=jnp.float32))
            emb_q = (parts[0] + parts[1]) + parts[2]
            t = emb_q - res_h
            emb = res_h + t
            stats_ref[row0:row0 + hb, l:l + 1] = jnp.sqrt(
                jnp.sum(emb * emb, axis=-1, keepdims=True))
            ql = ql + jnp.sum(t * t)
            res_h = res_h - emb
            zq = zq + emb
            key = key * n_codes + ids
        keys_ref[row0:row0 + hb, 0:1] = key
        return ql, zq, res_h

    nchain = 4
    hb = bb // nchain
    ql_sum = jnp.float32(0.0)
    zqs, rfs = [], []
    for ci in range(nchain):
        qlc, zqc, rfc = vq_chain(res[ci * hb:(ci + 1) * hb], ci * hb)
        ql_sum = ql_sum + qlc
        zqs.append(zqc)
        rfs.append(rfc)
    z_q = jnp.concatenate(zqs, axis=0)
    res = jnp.concatenate(rfs, axis=0)

    g = jnp.maximum(jnp.dot(z_q, wd0[...], preferred_element_type=jnp.float32)
                    + bd0[...], 0.0)
    g = jnp.maximum(jnp.dot(g, wd1[...], preferred_element_type=jnp.float32)
                    + bd1[...], 0.0)
    g = jnp.dot(g, wd2[...], preferred_element_type=jnp.float32) + bd2[...]
    for _ in range(2):
        nrm = jnp.sqrt(jnp.sum(g * g, axis=-1, keepdims=True))
        g = g / jnp.maximum(nrm, 1e-12)

    dx = g - x
    recon = jnp.sum(dx * dx, axis=-1, keepdims=True)
    rerr = jnp.sqrt(recon)
    stats_ref[:, 3:4] = rerr
    nx = jnp.sqrt(jnp.sum(x * x, axis=-1, keepdims=True))
    ng = jnp.sqrt(jnp.sum(g * g, axis=-1, keepdims=True))
    cos = jnp.sum(x * g, axis=-1, keepdims=True) / (nx * ng + 1e-8)
    dq = z_enc - z_q
    resn = jnp.sqrt(jnp.sum(dq * dq, axis=-1, keepdims=True))

    li = jax.lax.broadcasted_iota(jnp.int32, (1, 8), 1)
    vals = (jnp.where(li == 0, jnp.sum(recon), 0.0)
            + jnp.where(li == 1, ql_sum, 0.0)
            + jnp.where(li == 2, jnp.sum(cos), 0.0)
            + jnp.where(li == 3, jnp.sum(resn), 0.0))
    mvals = jnp.where(li == 0, jnp.min(rerr),
                      jnp.where(li == 1, -jnp.max(rerr), BIG))

    @pl.when(pid == 0)
    def _():
        acc_ref[...] = jnp.zeros_like(acc_ref)
        macc_ref[...] = jnp.full_like(macc_ref, BIG)

    acc_ref[...] += vals
    macc_ref[...] = jnp.minimum(macc_ref[...], mvals)


def _stats_block(k_row_ref, stats_ref, keys_ref, macc_ref,
                 out_ref, hacc_ref, *, rb, n, nb, targets):
    pid = pl.program_id(0)
    nblk = n // rb
    base = pid * rb
    k_col = keys_ref[pl.ds(base, rb), 0:1]
    irow = base + jax.lax.broadcasted_iota(jnp.int32, (rb, 1), 0)

    k_row = k_row_ref[...]
    jiota = jax.lax.broadcasted_iota(jnp.int32, (rb, n), 1)
    dup_after = jnp.logical_and(k_row == k_col, jiota > irow)
    has_dup = jnp.max(dup_after.astype(jnp.float32), axis=-1, keepdims=True)
    distinct_part = jnp.sum(1.0 - has_dup)

    e_col = stats_ref[pl.ds(base, rb), 3:4]
    mn = macc_ref[0, 0]
    mx = -macc_ref[0, 1]
    w = jnp.maximum((mx - mn) * (1.0 / nb), 1e-30)
    binid = jnp.clip(jnp.floor((e_col - mn) / w).astype(jnp.int32), 0, nb - 1)
    rid = binid // 128
    lid = binid - rid * 128
    arow = (jax.lax.broadcasted_iota(jnp.int32, (rb, nb // 128), 1)
            == rid).astype(jnp.bfloat16)
    alane = (jax.lax.broadcasted_iota(jnp.int32, (rb, 128), 1)
             == lid).astype(jnp.bfloat16)
    hist = jax.lax.dot_general(arow, alane, (((0,), (0,)), ((), ())),
                               preferred_element_type=jnp.float32)

    li = jax.lax.broadcasted_iota(jnp.int32, (1, 8), 1)

    @pl.when(pid == 0)
    def _():
        out_ref[...] = jnp.zeros_like(out_ref)
        hacc_ref[...] = jnp.zeros_like(hacc_ref)

    out_ref[...] += jnp.where(li == 6, distinct_part, 0.0)
    hacc_ref[...] += hist

    @pl.when(pid == nblk - 1)
    def _():
        h2 = hacc_ref[...]
        ut = (jax.lax.broadcasted_iota(jnp.int32, (128, 128), 0)
              <= jax.lax.broadcasted_iota(jnp.int32, (128, 128), 1)
              ).astype(jnp.float32)
        cum_row = jax.lax.dot_general(h2, ut, (((1,), (0,)), ((), ())),
                                      preferred_element_type=jnp.float32,
                                      precision=jax.lax.Precision.HIGHEST)
        tot = jnp.sum(h2, axis=1, keepdims=True)
        m = nb // 128
        st = (jax.lax.broadcasted_iota(jnp.int32, (m, m), 0)
              < jax.lax.broadcasted_iota(jnp.int32, (m, m), 1)
              ).astype(jnp.float32)
        prefix = jax.lax.dot_general(st, tot, (((0,), (0,)), ((), ())),
                                     preferred_element_type=jnp.float32,
                                     precision=jax.lax.Precision.HIGHEST)
        cum = cum_row + prefix
        qv = jnp.zeros((1, 8), jnp.float32)
        for slot, tgt in enumerate(targets):
            bidx = jnp.sum((cum <= float(tgt)).astype(jnp.float32))
            qv = qv + jnp.where(li == slot, bidx, 0.0)
        out_ref[...] += qv


def kernel(x, We0, be0, We1, be1, We2, be2, Wd0, bd0, Wd1, bd1, Wd2, bd2,
           codebooks, gumbel_t):
    B, D = x.shape
    L, K, E = codebooks.shape
    BB = 1024
    grid_a = B // BB

    be0r, be1r, be2r = be0[None, :], be1[None, :], be2[None, :]
    bd0r, bd1r, bd2r = bd0[None, :], bd1[None, :], bd2[None, :]

    full = lambda arr: pl.BlockSpec(arr.shape, lambda i: (0,) * arr.ndim)
    fwd = functools.partial(_fwd_block, n_levels=L, n_codes=K)
    stats, keys, acc, macc = pl.pallas_call(
        fwd,
        grid=(grid_a,),
        in_specs=[
            pl.BlockSpec((BB, D), lambda i: (i, 0)),
            full(We0), full(be0r), full(We1), full(be1r),
            full(We2), full(be2r),
            full(Wd0), full(bd0r), full(Wd1), full(bd1r),
            full(Wd2), full(bd2r),
            full(codebooks),
        ],
        out_specs=[
            pl.BlockSpec((BB, 8), lambda i: (i, 0)),
            pl.BlockSpec((BB, 8), lambda i: (i, 0)),
            pl.BlockSpec((1, 8), lambda i: (0, 0)),
            pl.BlockSpec((1, 8), lambda i: (0, 0)),
        ],
        out_shape=[
            jax.ShapeDtypeStruct((B, 8), jnp.float32),
            jax.ShapeDtypeStruct((B, 8), jnp.int32),
            jax.ShapeDtypeStruct((1, 8), jnp.float32),
            jax.ShapeDtypeStruct((1, 8), jnp.float32),
        ],
        scratch_shapes=[
            pltpu.VMEM((L, K, E), jnp.bfloat16),
            pltpu.VMEM((L, K, E), jnp.bfloat16),
            pltpu.VMEM((L, K, E), jnp.bfloat16),
        ],
    )(x, We0, be0r, We1, be1r, We2, be2r,
      Wd0, bd0r, Wd1, bd1r, Wd2, bd2r, codebooks)

    qspec = []
    ranks = []
    for q in (0.5, 0.9, 0.99):
        pos = q * (B - 1)
        lo_r = int(math.floor(pos))
        frac = pos - lo_r
        qspec.append(frac)
        ranks.extend([lo_r, lo_r + 1])

    k_row = keys[:, 0].reshape(1, B)
    RB = 1024
    sb = functools.partial(_stats_block, rb=RB, n=B, nb=NBINS,
                           targets=tuple(ranks))
    qacc, _hist = pl.pallas_call(
        sb,
        grid=(B // RB,),
        in_specs=[full(k_row), full(stats), full(keys), full(macc)],
        out_specs=[pl.BlockSpec((1, 8), lambda i: (0, 0)),
                   pl.BlockSpec((NBINS // 128, 128), lambda i: (0, 0))],
        out_shape=[jax.ShapeDtypeStruct((1, 8), jnp.float32),
                   jax.ShapeDtypeStruct((NBINS // 128, 128), jnp.float32)],
    )(k_row, stats, keys, macc)

    s_recon, s_ql, s_cos, s_resn = acc[0, 0], acc[0, 1], acc[0, 2], acc[0, 3]
    mean_recon = s_recon / B
    rq_l = (1.0 + COMMIT_W) * s_ql / B
    loss = mean_recon + rq_l
    cosine_sim = s_cos / B
    rmse = jnp.sqrt(s_recon / (B * D))
    quantization_error = s_resn / B
    embs_norm = stats[:, :L].T
    p_unique_ids = qacc[0, 6] / B

    mn = macc[0, 0]
    mx = -macc[0, 1]
    w = jnp.maximum((mx - mn) * (1.0 / NBINS), 1e-30)
    qs = []
    for i, frac in enumerate(qspec):
        v_lo = mn + w * (qacc[0, 2 * i] + 0.5)
        v_hi = mn + w * (qacc[0, 2 * i + 1] + 0.5)
        qs.append(v_lo * (1.0 - jnp.float32(frac)) + v_hi * jnp.float32(frac))
    p50, p90, p99 = qs

    return (loss, mean_recon, rq_l, embs_norm, p_unique_ids, cosine_sim,
            rmse, quantization_error, p50, p90, p99)


# speedup vs baseline: 1.2539x; 1.0066x over previous
import functools
import math

import jax
import jax.numpy as jnp
from jax.experimental import pallas as pl
from jax.experimental.pallas import tpu as pltpu


COMMIT_W = 0.25
NBINS = 2048
BIG = 3.0e38


def _fwd_block(x_ref, we0, be0, we1, be1, we2, be2,
               wd0, bd0, wd1, bd1, wd2, bd2, cb_ref,
               stats_ref, keys_ref, acc_ref, macc_ref,
               cbh_ref, cbm_ref, cbl_ref,
               *, n_levels, n_codes):
    pid = pl.program_id(0)
    x = x_ref[...]
    bb = x.shape[0]

    @pl.when(pid == 0)
    def _():
        cbf = cb_ref[...]
        u = jax.lax.bitcast_convert_type(cbf, jnp.uint32)
        hi = jax.lax.bitcast_convert_type(u & jnp.uint32(0xFFFF0000),
                                          jnp.float32)
        r1 = cbf - hi
        u1 = jax.lax.bitcast_convert_type(r1, jnp.uint32)
        mid = jax.lax.bitcast_convert_type(u1 & jnp.uint32(0xFFFF0000),
                                           jnp.float32)
        cbh_ref[...] = hi.astype(jnp.bfloat16)
        cbm_ref[...] = mid.astype(jnp.bfloat16)
        cbl_ref[...] = (r1 - mid).astype(jnp.bfloat16)

    h = jnp.maximum(jnp.dot(x, we0[...], preferred_element_type=jnp.float32)
                    + be0[...], 0.0)
    h = jnp.maximum(jnp.dot(h, we1[...], preferred_element_type=jnp.float32)
                    + be1[...], 0.0)
    res = jnp.dot(h, we2[...], preferred_element_type=jnp.float32) + be2[...]
    z_enc = res

    def vq_chain(res_h, row0):
        hb = res_h.shape[0]
        iota = jax.lax.broadcasted_iota(jnp.int32, (hb, n_codes), 1)
        key = jnp.zeros((hb, 1), jnp.int32)
        ql = jnp.float32(0.0)
        zq = jnp.zeros_like(res_h)
        for l in range(n_levels):
            cb = cb_ref[l]
            rn = jnp.sum(res_h * res_h, axis=-1, keepdims=True)
            scores = jax.lax.dot_general(res_h, cb, (((1,), (1,)), ((), ())),
                                         preferred_element_type=jnp.float32)
            cn = jnp.sum(cb * cb, axis=-1)[None, :]
            d = rn - 2.0 * scores + cn
            ids = jnp.argmin(d, axis=-1, keepdims=True).astype(jnp.int32)
            oh = (iota == ids).astype(jnp.bfloat16)
            parts = []
            for part_ref in (cbh_ref, cbm_ref, cbl_ref):
                parts.append(jax.lax.dot_general(
                    oh, part_ref[l], (((1,), (0,)), ((), ())),
                    preferred_element_type=jnp.float32))
            emb_q = (parts[0] + parts[1]) + parts[2]
            t = emb_q - res_h
            emb = res_h + t
            stats_ref[row0:row0 + hb, l:l + 1] = jnp.sqrt(
                jnp.sum(emb * emb, axis=-1, keepdims=True))
            ql = ql + jnp.sum(t * t)
            res_h = res_h - emb
            zq = zq + emb
            key = key * n_codes + ids
        keys_ref[row0:row0 + hb, 0:1] = key
        return ql, zq, res_h

    nchain = 4
    hb = bb // nchain
    ql_sum = jnp.float32(0.0)
    zqs, rfs = [], []
    for ci in range(nchain):
        qlc, zqc, rfc = vq_chain(res[ci * hb:(ci + 1) * hb], ci * hb)
        ql_sum = ql_sum + qlc
        zqs.append(zqc)
        rfs.append(rfc)
    z_q = jnp.concatenate(zqs, axis=0)
    res = jnp.concatenate(rfs, axis=0)

    g = jnp.maximum(jnp.dot(z_q, wd0[...], preferred_element_type=jnp.float32)
                    + bd0[...], 0.0)
    g = jnp.maximum(jnp.dot(g, wd1[...], preferred_element_type=jnp.float32)
                    + bd1[...], 0.0)
    g = jnp.dot(g, wd2[...], preferred_element_type=jnp.float32) + bd2[...]
    for _ in range(2):
        nrm = jnp.sqrt(jnp.sum(g * g, axis=-1, keepdims=True))
        g = g / jnp.maximum(nrm, 1e-12)

    dx = g - x
    recon = jnp.sum(dx * dx, axis=-1, keepdims=True)
    rerr = jnp.sqrt(recon)
    stats_ref[:, 3:4] = rerr
    nx = jnp.sqrt(jnp.sum(x * x, axis=-1, keepdims=True))
    ng = jnp.sqrt(jnp.sum(g * g, axis=-1, keepdims=True))
    cos = jnp.sum(x * g, axis=-1, keepdims=True) / (nx * ng + 1e-8)
    dq = z_enc - z_q
    resn = jnp.sqrt(jnp.sum(dq * dq, axis=-1, keepdims=True))

    li = jax.lax.broadcasted_iota(jnp.int32, (1, 8), 1)
    vals = (jnp.where(li == 0, jnp.sum(recon), 0.0)
            + jnp.where(li == 1, ql_sum, 0.0)
            + jnp.where(li == 2, jnp.sum(cos), 0.0)
            + jnp.where(li == 3, jnp.sum(resn), 0.0))
    mvals = jnp.where(li == 0, jnp.min(rerr),
                      jnp.where(li == 1, -jnp.max(rerr), BIG))

    @pl.when(pid == 0)
    def _():
        acc_ref[...] = jnp.zeros_like(acc_ref)
        macc_ref[...] = jnp.full_like(macc_ref, BIG)

    acc_ref[...] += vals
    macc_ref[...] = jnp.minimum(macc_ref[...], mvals)


def _stats_block(k_row_ref, stats_ref, keys_ref, macc_ref,
                 out_ref, hacc_ref, *, rb, n, nb, targets):
    pid = pl.program_id(0)
    nblk = n // rb
    base = pid * rb
    k_col = keys_ref[pl.ds(base, rb), 0:1]
    irow = base + jax.lax.broadcasted_iota(jnp.int32, (rb, 1), 0)

    k_row = k_row_ref[...]
    jiota = jax.lax.broadcasted_iota(jnp.int32, (rb, n), 1)
    dup_after = jnp.logical_and(k_row == k_col, jiota > irow)
    has_dup = jnp.max(dup_after.astype(jnp.float32), axis=-1, keepdims=True)
    distinct_part = jnp.sum(1.0 - has_dup)

    e_col = stats_ref[pl.ds(base, rb), 3:4]
    mn = macc_ref[0, 0]
    mx = -macc_ref[0, 1]
    w = jnp.maximum((mx - mn) * (1.0 / nb), 1e-30)
    binid = jnp.clip(jnp.floor((e_col - mn) / w).astype(jnp.int32), 0, nb - 1)
    rid = binid // 128
    lid = binid - rid * 128
    arow = (jax.lax.broadcasted_iota(jnp.int32, (rb, nb // 128), 1)
            == rid).astype(jnp.bfloat16)
    alane = (jax.lax.broadcasted_iota(jnp.int32, (rb, 128), 1)
             == lid).astype(jnp.bfloat16)
    hist = jax.lax.dot_general(arow, alane, (((0,), (0,)), ((), ())),
                               preferred_element_type=jnp.float32)

    li = jax.lax.broadcasted_iota(jnp.int32, (1, 8), 1)

    @pl.when(pid == 0)
    def _():
        out_ref[...] = jnp.zeros_like(out_ref)
        hacc_ref[...] = jnp.zeros_like(hacc_ref)

    out_ref[...] += jnp.where(li == 6, distinct_part, 0.0)
    hacc_ref[...] += hist

    @pl.when(pid == nblk - 1)
    def _():
        h2 = hacc_ref[...]
        ut = (jax.lax.broadcasted_iota(jnp.int32, (128, 128), 0)
              <= jax.lax.broadcasted_iota(jnp.int32, (128, 128), 1)
              ).astype(jnp.float32)
        cum_row = jax.lax.dot_general(h2, ut, (((1,), (0,)), ((), ())),
                                      preferred_element_type=jnp.float32,
                                      precision=jax.lax.Precision.HIGHEST)
        tot = jnp.sum(h2, axis=1, keepdims=True)
        m = nb // 128
        st = (jax.lax.broadcasted_iota(jnp.int32, (m, m), 0)
              < jax.lax.broadcasted_iota(jnp.int32, (m, m), 1)
              ).astype(jnp.float32)
        prefix = jax.lax.dot_general(st, tot, (((0,), (0,)), ((), ())),
                                     preferred_element_type=jnp.float32,
                                     precision=jax.lax.Precision.HIGHEST)
        cum = cum_row + prefix
        qv = jnp.zeros((1, 8), jnp.float32)
        for slot, tgt in enumerate(targets):
            bidx = jnp.sum((cum <= float(tgt)).astype(jnp.float32))
            qv = qv + jnp.where(li == slot, bidx, 0.0)
        out_ref[...] += qv


def kernel(x, We0, be0, We1, be1, We2, be2, Wd0, bd0, Wd1, bd1, Wd2, bd2,
           codebooks, gumbel_t):
    B, D = x.shape
    L, K, E = codebooks.shape
    BB = 1024
    grid_a = B // BB

    be0r, be1r, be2r = be0[None, :], be1[None, :], be2[None, :]
    bd0r, bd1r, bd2r = bd0[None, :], bd1[None, :], bd2[None, :]

    full = lambda arr: pl.BlockSpec(arr.shape, lambda i: (0,) * arr.ndim)
    fwd = functools.partial(_fwd_block, n_levels=L, n_codes=K)
    stats, keys, acc, macc = pl.pallas_call(
        fwd,
        grid=(grid_a,),
        in_specs=[
            pl.BlockSpec((BB, D), lambda i: (i, 0)),
            full(We0), full(be0r), full(We1), full(be1r),
            full(We2), full(be2r),
            full(Wd0), full(bd0r), full(Wd1), full(bd1r),
            full(Wd2), full(bd2r),
            full(codebooks),
        ],
        out_specs=[
            pl.BlockSpec((BB, 8), lambda i: (i, 0)),
            pl.BlockSpec((BB, 8), lambda i: (i, 0)),
            pl.BlockSpec((1, 8), lambda i: (0, 0)),
            pl.BlockSpec((1, 8), lambda i: (0, 0)),
        ],
        out_shape=[
            jax.ShapeDtypeStruct((B, 8), jnp.float32),
            jax.ShapeDtypeStruct((B, 8), jnp.int32),
            jax.ShapeDtypeStruct((1, 8), jnp.float32),
            jax.ShapeDtypeStruct((1, 8), jnp.float32),
        ],
        scratch_shapes=[
            pltpu.VMEM((L, K, E), jnp.bfloat16),
            pltpu.VMEM((L, K, E), jnp.bfloat16),
            pltpu.VMEM((L, K, E), jnp.bfloat16),
        ],
    )(x, We0, be0r, We1, be1r, We2, be2r,
      Wd0, bd0r, Wd1, bd1r, Wd2, bd2r, codebooks)

    qspec = []
    ranks = []
    for q in (0.5, 0.9, 0.99):
        pos = q * (B - 1)
        lo_r = int(math.floor(pos))
        frac = pos - lo_r
        qspec.append(frac)
        ranks.extend([lo_r, lo_r + 1])

    k_row = keys[:, 0].reshape(1, B)
    RB = 2048
    sb = functools.partial(_stats_block, rb=RB, n=B, nb=NBINS,
                           targets=tuple(ranks))
    qacc, _hist = pl.pallas_call(
        sb,
        grid=(B // RB,),
        in_specs=[full(k_row), full(stats), full(keys), full(macc)],
        out_specs=[pl.BlockSpec((1, 8), lambda i: (0, 0)),
                   pl.BlockSpec((NBINS // 128, 128), lambda i: (0, 0))],
        out_shape=[jax.ShapeDtypeStruct((1, 8), jnp.float32),
                   jax.ShapeDtypeStruct((NBINS // 128, 128), jnp.float32)],
    )(k_row, stats, keys, macc)

    s_recon, s_ql, s_cos, s_resn = acc[0, 0], acc[0, 1], acc[0, 2], acc[0, 3]
    mean_recon = s_recon / B
    rq_l = (1.0 + COMMIT_W) * s_ql / B
    loss = mean_recon + rq_l
    cosine_sim = s_cos / B
    rmse = jnp.sqrt(s_recon / (B * D))
    quantization_error = s_resn / B
    embs_norm = stats[:, :L].T
    p_unique_ids = qacc[0, 6] / B

    mn = macc[0, 0]
    mx = -macc[0, 1]
    w = jnp.maximum((mx - mn) * (1.0 / NBINS), 1e-30)
    qs = []
    for i, frac in enumerate(qspec):
        v_lo = mn + w * (qacc[0, 2 * i] + 0.5)
        v_hi = mn + w * (qacc[0, 2 * i + 1] + 0.5)
        qs.append(v_lo * (1.0 - jnp.float32(frac)) + v_hi * jnp.float32(frac))
    p50, p90, p99 = qs

    return (loss, mean_recon, rq_l, embs_norm, p_unique_ids, cosine_sim,
            rmse, quantization_error, p50, p90, p99)
